# baseline (speedup 1.0000x reference)
import jax
import jax.numpy as jnp
from jax import lax
from jax.experimental import pallas as pl
from jax.experimental.pallas import tpu as pltpu
from jax.experimental.pallas import tpu_sc as plsc

N = 10000
E = 320000
D = 128
DH = D // 2
NC = 2
NS = 16
NW = NC * NS
EPT = E // NS
CH = 80
NCH = EPT // CH
NBUF = 5
GAHEAD = 4
SLAG = NBUF - GAHEAD
PR = 80 // CH
NP = 10240
RPT = NP // NS
ZR = 32
ZCR = 160
CNTW = 16

_f32 = jnp.float32


def _make_agg_body(with_cnt):
    def _agg_body(*refs):
        it = iter(refs)
        xs_hbm = next(it); src_hbm = next(it); dst_hbm = next(it)
        agg_out = next(it)
        cnt_out = next(it) if with_cnt else None
        src_all = next(it); dst_all = next(it)
        rows = tuple(next(it) for _ in range(NBUF))
        zbuf = next(it)
        if with_cnt:
            zcnt = next(it); ebuf = next(it)
        gsem = next(it); ssem = next(it)
        csem = next(it) if with_cnt else None
        psem = next(it)
        agg_sh = next(it)
        cnt_sh = next(it) if with_cnt else None
        c = lax.axis_index("c")
        s = lax.axis_index("s")

        pfs = pltpu.async_copy(src_hbm.at[pl.ds(s * (EPT // 80), EPT // 80)],
                               src_all, psem)
        pfd = pltpu.async_copy(dst_hbm.at[pl.ds(s * NCH, NCH)], dst_all, psem)

        zrow = jnp.zeros((16,), _f32)

        def zb_body(r, carry):
            for j in range(DH // 16):
                zbuf[r, pl.ds(j * 16, 16)] = zrow
            return carry
        lax.fori_loop(0, ZR, zb_body, 0)

        rbase = s * RPT
        zd = [pltpu.async_copy(zbuf, agg_sh.at[pl.ds(rbase + j * ZR, ZR)], ssem)
              for j in range(RPT // ZR)]
        if with_cnt:
            def zc_body(r, carry):
                zcnt[r, :] = zrow
                return carry
            lax.fori_loop(0, ZCR, zc_body, 0)

            ehot = jnp.where(lax.iota(jnp.int32, 16) == 0, 1.0, 0.0)

            def eb_body(r, carry):
                ebuf[r, :] = ehot.astype(_f32)
                return carry
            lax.fori_loop(0, CH, eb_body, 0)

            zd += [pltpu.async_copy(
                zcnt, cnt_sh.at[pl.ds(rbase + j * ZCR, ZCR)], ssem)
                for j in range(RPT // ZCR)]
        for d in zd:
            d.wait()

        pfs.wait()
        pfd.wait()

        def off_body(r, carry):
            for k in range(80 // 16):
                src_all[r, pl.ds(k * 16, 16)] = (
                    src_all[r, pl.ds(k * 16, 16)] * 2 + c)
            return carry
        lax.fori_loop(0, EPT // 80, off_body, 0)

        plsc.subcore_barrier()

        def gather_desc(i, b, half):
            return pltpu.make_async_copy(
                xs_hbm.at[src_all.at[i // PR, pl.ds(half * CH, CH)]],
                rows[b], gsem)

        def scatter_desc(i, b):
            return pltpu.make_async_copy(
                rows[b], agg_sh.at[dst_all.at[i]], ssem)

        for b in range(GAHEAD):
            gather_desc(b, b, b % PR).start()

        def round_body(j, carry):
            for b in range(NBUF):
                i = j * NBUF + b
                gather_desc(i, b, b % PR).wait()
                pltpu.async_copy(rows[b], agg_sh.at[dst_all.at[i]], ssem,
                                 add=True)
                if with_cnt:
                    @pl.when((i % 2) == c)
                    def _():
                        pltpu.async_copy(ebuf, cnt_sh.at[dst_all.at[i]],
                                         csem, add=True)

                        @pl.when(i >= 2)
                        def _():
                            pltpu.make_async_copy(
                                ebuf, cnt_sh.at[dst_all.at[i]], csem).wait()

                @pl.when(i >= SLAG)
                def _():
                    scatter_desc(i, b).wait()

                @pl.when(i + GAHEAD < NCH)
                def _():
                    gather_desc(i + GAHEAD, (b + GAHEAD) % NBUF,
                                (b + GAHEAD) % PR).start()
            return carry
        lax.fori_loop(0, NCH // NBUF, round_body, 0)

        for _ in range(SLAG):
            scatter_desc(0, 0).wait()
        if with_cnt:
            pltpu.make_async_copy(ebuf, cnt_sh.at[dst_all.at[0]], csem).wait()

        plsc.subcore_barrier()

        obase = c * NP + rbase
        pltpu.sync_copy(agg_sh.at[pl.ds(rbase, RPT)],
                        agg_out.at[pl.ds(obase, RPT), pl.ds(0, DH)])
        if with_cnt:
            pltpu.sync_copy(cnt_sh.at[pl.ds(rbase, RPT)],
                            cnt_out.at[pl.ds(obase, RPT), pl.ds(0, CNTW)])
    return _agg_body


def _build_sc_agg(with_cnt):
    out_type = [jax.ShapeDtypeStruct((NC * NP, D), _f32)]
    if with_cnt:
        out_type.append(jax.ShapeDtypeStruct((NC * NP, D), _f32))
    scratch = [
        pltpu.VMEM((EPT // 80, 80), jnp.int32),
        pltpu.VMEM((NCH, CH), jnp.int32),
    ]
    scratch += [pltpu.VMEM((CH, DH), _f32) for _ in range(NBUF)]
    scratch += [pltpu.VMEM((ZR, DH), _f32)]
    if with_cnt:
        scratch += [pltpu.VMEM((ZCR, CNTW), _f32),
                    pltpu.VMEM((CH, CNTW), _f32)]
    scratch += [pltpu.SemaphoreType.DMA,
                pltpu.SemaphoreType.DMA]
    if with_cnt:
        scratch += [pltpu.SemaphoreType.DMA]
    scratch += [pltpu.SemaphoreType.DMA,
                pltpu.VMEM_SHARED((NP, DH), _f32)]
    if with_cnt:
        scratch += [pltpu.VMEM_SHARED((NP, CNTW), _f32)]
    return pl.kernel(
        _make_agg_body(with_cnt),
        out_type=tuple(out_type) if with_cnt else out_type[0],
        mesh=plsc.VectorSubcoreMesh(core_axis_name="c", subcore_axis_name="s"),
        compiler_params=pltpu.CompilerParams(use_tc_tiling_on_sc=False),
        scratch_types=scratch,
    )


_sc_agg_cnt = _build_sc_agg(True)
_sc_agg = _build_sc_agg(False)


def _mean_from_parts(agg_ref, cnt_ref):
    cnt = jnp.sum(cnt_ref[0, :, :CNTW] + cnt_ref[1, :, :CNTW],
                  axis=1, keepdims=True)
    inv = 1.0 / jnp.maximum(cnt, 1.0)
    return agg_ref[0, :, :DH] * inv, agg_ref[1, :, :DH] * inv


def _sage_out(m0, m1, xd, wl_ref, bl_ref, wr_ref):
    out = lax.dot_general(m0, wl_ref[:, :DH], (((1,), (1,)), ((), ())),
                          preferred_element_type=_f32)
    out = out + lax.dot_general(m1, wl_ref[:, DH:], (((1,), (1,)), ((), ())),
                                preferred_element_type=_f32)
    out = out + lax.dot_general(xd, wr_ref[...], (((1,), (1,)), ((), ())),
                                preferred_element_type=_f32)
    out = out + bl_ref[...]
    return jnp.maximum(out, 0.0)


def _tc_body(agg_ref, cnt_ref, x_ref, wl_ref, bl_ref, wr_ref, o_ref):
    m0, m1 = _mean_from_parts(agg_ref, cnt_ref)
    o_ref[...] = _sage_out(m0, m1, x_ref[...], wl_ref, bl_ref, wr_ref)


R = 5000


def _tc_layer(agg, cnt, x, Wl, bl, Wr):
    return pl.pallas_call(
        _tc_body,
        grid=(N // R,),
        in_specs=[
            pl.BlockSpec((NC, R, D), lambda i: (0, i, 0)),
            pl.BlockSpec((NC, R, D), lambda i: (0, i, 0)),
            pl.BlockSpec((R, D), lambda i: (i, 0)),
            pl.BlockSpec((D, D), lambda i: (0, 0)),
            pl.BlockSpec((1, D), lambda i: (0, 0)),
            pl.BlockSpec((D, D), lambda i: (0, 0)),
        ],
        out_specs=pl.BlockSpec((R, D), lambda i: (i, 0)),
        out_shape=jax.ShapeDtypeStruct((N, D), _f32),
    )(agg, cnt, x, Wl, bl, Wr)


def kernel(x, edge_index, batch, Wl1, bl1, Wr1, Wl2, bl2, Wr2):
    src = edge_index[0].reshape(E // 80, 80)
    dst = edge_index[1].reshape(E // CH, CH)
    agg1, cnt1 = _sc_agg_cnt(x.reshape(NC * N, DH), src, dst)
    agg1 = agg1.reshape(NC, NP, D)
    cnt1 = cnt1.reshape(NC, NP, D)
    h = _tc_layer(agg1, cnt1, x, Wl1, bl1.reshape(1, D), Wr1)
    agg2 = _sc_agg(h.reshape(NC * N, DH), src, dst).reshape(NC, NP, D)
    out = _tc_layer(agg2, cnt1, h, Wl2, bl2.reshape(1, D), Wr2)
    return out


# speedup vs baseline: 1.0050x; 1.0050x over previous
import jax
import jax.numpy as jnp
from jax import lax
from jax.experimental import pallas as pl
from jax.experimental.pallas import tpu as pltpu
from jax.experimental.pallas import tpu_sc as plsc

N = 10000
E = 320000
D = 128
DH = D // 2
NC = 2
NS = 16
NW = NC * NS
EPT = E // NS
CH = 80
NCH = EPT // CH
NBUF = 5
GAHEAD = 4
SLAG = NBUF - GAHEAD
PR = 80 // CH
NP = 10240
RPT = NP // NS
ZR = 32
ZCR = 160
CNTW = 16

_f32 = jnp.float32


def _make_agg_body(with_cnt):
    def _agg_body(*refs):
        it = iter(refs)
        xs_hbm = next(it); src_hbm = next(it); dst_hbm = next(it)
        agg_out = next(it)
        cnt_out = next(it) if with_cnt else None
        src_all = next(it); dst_all = next(it)
        rows = tuple(next(it) for _ in range(NBUF))
        zbuf = next(it)
        if with_cnt:
            zcnt = next(it); ebuf = next(it)
        gsem = next(it); ssem = next(it)
        csem = next(it) if with_cnt else None
        psem = next(it)
        agg_sh = next(it)
        cnt_sh = next(it) if with_cnt else None
        c = lax.axis_index("c")
        s = lax.axis_index("s")

        pfs = pltpu.async_copy(src_hbm.at[pl.ds(s * (EPT // 80), EPT // 80)],
                               src_all, psem)
        pfd = pltpu.async_copy(dst_hbm.at[pl.ds(s * NCH, NCH)], dst_all, psem)

        zrow = jnp.zeros((16,), _f32)

        def zb_body(r, carry):
            for j in range(DH // 16):
                zbuf[r, pl.ds(j * 16, 16)] = zrow
            return carry
        lax.fori_loop(0, ZR, zb_body, 0)

        rbase = s * RPT
        zd = [pltpu.async_copy(zbuf, agg_sh.at[pl.ds(rbase + j * ZR, ZR)], ssem)
              for j in range(RPT // ZR)]
        if with_cnt:
            def zc_body(r, carry):
                zcnt[r, :] = zrow
                return carry
            lax.fori_loop(0, ZCR, zc_body, 0)

            ehot = jnp.where(lax.iota(jnp.int32, 16) == 0, 1.0, 0.0)

            def eb_body(r, carry):
                ebuf[r, :] = ehot.astype(_f32)
                return carry
            lax.fori_loop(0, CH, eb_body, 0)

            zd += [pltpu.async_copy(
                zcnt, cnt_sh.at[pl.ds(rbase + j * ZCR, ZCR)], ssem)
                for j in range(RPT // ZCR)]
        for d in zd:
            d.wait()

        pfs.wait()
        pfd.wait()

        def off_body(r, carry):
            for k in range(80 // 16):
                src_all[r, pl.ds(k * 16, 16)] = (
                    src_all[r, pl.ds(k * 16, 16)] * 2 + c)
            return carry
        lax.fori_loop(0, EPT // 80, off_body, 0)

        plsc.subcore_barrier()

        def gather_desc(i, b, half):
            return pltpu.make_async_copy(
                xs_hbm.at[src_all.at[i // PR, pl.ds(half * CH, CH)]],
                rows[b], gsem)

        def scatter_desc(i, b):
            return pltpu.make_async_copy(
                rows[b], agg_sh.at[dst_all.at[i]], ssem)

        for b in range(GAHEAD):
            gather_desc(b, b, b % PR).start()

        def round_body(j, carry):
            for b in range(NBUF):
                i = j * NBUF + b
                gather_desc(i, b, b % PR).wait()
                pltpu.async_copy(rows[b], agg_sh.at[dst_all.at[i]], ssem,
                                 add=True)
                if with_cnt:
                    @pl.when((i % 2) == c)
                    def _():
                        pltpu.async_copy(ebuf, cnt_sh.at[dst_all.at[i]],
                                         csem, add=True)

                        @pl.when(i >= 2)
                        def _():
                            pltpu.make_async_copy(
                                ebuf, cnt_sh.at[dst_all.at[i]], csem).wait()

                @pl.when(i >= SLAG)
                def _():
                    scatter_desc(i, b).wait()

                @pl.when(i + GAHEAD < NCH)
                def _():
                    gather_desc(i + GAHEAD, (b + GAHEAD) % NBUF,
                                (b + GAHEAD) % PR).start()
            return carry
        lax.fori_loop(0, NCH // NBUF, round_body, 0)

        for _ in range(SLAG):
            scatter_desc(0, 0).wait()
        if with_cnt:
            pltpu.make_async_copy(ebuf, cnt_sh.at[dst_all.at[0]], csem).wait()

        plsc.subcore_barrier()

        obase = c * NP + rbase
        pltpu.sync_copy(agg_sh.at[pl.ds(rbase, RPT)],
                        agg_out.at[pl.ds(obase, RPT), pl.ds(0, DH)])
        if with_cnt:
            pltpu.sync_copy(cnt_sh.at[pl.ds(rbase, RPT)],
                            cnt_out.at[pl.ds(obase, RPT), pl.ds(0, CNTW)])
    return _agg_body


def _build_sc_agg(with_cnt):
    out_type = [jax.ShapeDtypeStruct((NC * NP, D), _f32)]
    if with_cnt:
        out_type.append(jax.ShapeDtypeStruct((NC * NP, D), _f32))
    scratch = [
        pltpu.VMEM((EPT // 80, 80), jnp.int32),
        pltpu.VMEM((NCH, CH), jnp.int32),
    ]
    scratch += [pltpu.VMEM((CH, DH), _f32) for _ in range(NBUF)]
    scratch += [pltpu.VMEM((ZR, DH), _f32)]
    if with_cnt:
        scratch += [pltpu.VMEM((ZCR, CNTW), _f32),
                    pltpu.VMEM((CH, CNTW), _f32)]
    scratch += [pltpu.SemaphoreType.DMA,
                pltpu.SemaphoreType.DMA]
    if with_cnt:
        scratch += [pltpu.SemaphoreType.DMA]
    scratch += [pltpu.SemaphoreType.DMA,
                pltpu.VMEM_SHARED((NP, DH), _f32)]
    if with_cnt:
        scratch += [pltpu.VMEM_SHARED((NP, CNTW), _f32)]
    return pl.kernel(
        _make_agg_body(with_cnt),
        out_type=tuple(out_type) if with_cnt else out_type[0],
        mesh=plsc.VectorSubcoreMesh(core_axis_name="c", subcore_axis_name="s"),
        compiler_params=pltpu.CompilerParams(use_tc_tiling_on_sc=False),
        scratch_types=scratch,
    )


_sc_agg_cnt = _build_sc_agg(True)
_sc_agg = _build_sc_agg(False)


def _mean_from_parts(agg_ref, cnt_ref):
    cnt = jnp.sum(cnt_ref[0, :, :CNTW] + cnt_ref[1, :, :CNTW],
                  axis=1, keepdims=True)
    inv = 1.0 / jnp.maximum(cnt, 1.0)
    return agg_ref[0, :, :DH] * inv, agg_ref[1, :, :DH] * inv


def _sage_out(m0, m1, xd, wl_ref, bl_ref, wr_ref):
    out = lax.dot_general(m0, wl_ref[:, :DH], (((1,), (1,)), ((), ())),
                          preferred_element_type=_f32)
    out = out + lax.dot_general(m1, wl_ref[:, DH:], (((1,), (1,)), ((), ())),
                                preferred_element_type=_f32)
    out = out + lax.dot_general(xd, wr_ref[...], (((1,), (1,)), ((), ())),
                                preferred_element_type=_f32)
    out = out + bl_ref[...]
    return jnp.maximum(out, 0.0)


def _tc_body(agg_ref, cnt_ref, x_ref, wl_ref, bl_ref, wr_ref, o_ref):
    m0, m1 = _mean_from_parts(agg_ref, cnt_ref)
    o_ref[...] = _sage_out(m0, m1, x_ref[...], wl_ref, bl_ref, wr_ref)


R = 2000


def _tc_layer(agg, cnt, x, Wl, bl, Wr):
    return pl.pallas_call(
        _tc_body,
        grid=(N // R,),
        in_specs=[
            pl.BlockSpec((NC, R, D), lambda i: (0, i, 0)),
            pl.BlockSpec((NC, R, D), lambda i: (0, i, 0)),
            pl.BlockSpec((R, D), lambda i: (i, 0)),
            pl.BlockSpec((D, D), lambda i: (0, 0)),
            pl.BlockSpec((1, D), lambda i: (0, 0)),
            pl.BlockSpec((D, D), lambda i: (0, 0)),
        ],
        out_specs=pl.BlockSpec((R, D), lambda i: (i, 0)),
        out_shape=jax.ShapeDtypeStruct((N, D), _f32),
    )(agg, cnt, x, Wl, bl, Wr)


def kernel(x, edge_index, batch, Wl1, bl1, Wr1, Wl2, bl2, Wr2):
    src = edge_index[0].reshape(E // 80, 80)
    dst = edge_index[1].reshape(E // CH, CH)
    agg1, cnt1 = _sc_agg_cnt(x.reshape(NC * N, DH), src, dst)
    agg1 = agg1.reshape(NC, NP, D)
    cnt1 = cnt1.reshape(NC, NP, D)
    h = _tc_layer(agg1, cnt1, x, Wl1, bl1.reshape(1, D), Wr1)
    agg2 = _sc_agg(h.reshape(NC * N, DH), src, dst).reshape(NC, NP, D)
    out = _tc_layer(agg2, cnt1, h, Wl2, bl2.reshape(1, D), Wr2)
    return out
